# Initial kernel scaffold; baseline (speedup 1.0000x reference)
import jax
import jax.numpy as jnp
from jax.experimental import pallas as pl


def kernel(atom_specific_values, index):
    raise NotImplementedError("write your pallas kernel here")


# speedup vs baseline: 29.2383x; 29.2383x over previous
import jax
import jax.numpy as jnp
from jax import lax
from jax.experimental import pallas as pl
from jax.experimental.pallas import tpu as pltpu, tpu_sc as plsc
import functools

N = 6400000
NSEG = 100000
NC = 2
NS = 16
CHUNK_ELEMS = 16000
NCHUNKS = N // CHUNK_ELEMS
MAX_CHUNKS_PER_W = -(-NCHUNKS // (NC * NS))
SEG_PAD = 100096
SLICE = SEG_PAD // NS


def _sc_body(vals_hbm, idx_hbm, out_hbm, vbuf, ibuf, acc, zbuf):
    c = lax.axis_index("c")
    s = lax.axis_index("s")
    wid = s * NC + c

    def zfill(i, _):
        zbuf[pl.ds(i * 16, 16)] = jnp.zeros((16,), jnp.float32)
        return 0
    lax.fori_loop(0, SLICE // 16, zfill, 0)
    pltpu.sync_copy(zbuf, acc.at[pl.ds(s * SLICE, SLICE)])
    plsc.subcore_barrier()

    def chunk_body(k, _):
        cid = wid + (NC * NS) * k

        @pl.when(cid < NCHUNKS)
        def _():
            el0 = cid * CHUNK_ELEMS
            pltpu.sync_copy(vals_hbm.at[pl.ds(el0, CHUNK_ELEMS)], vbuf)
            pltpu.sync_copy(idx_hbm.at[pl.ds(el0, CHUNK_ELEMS)], ibuf)
            pltpu.sync_copy(vbuf, acc.at[ibuf], add=True)
        return 0
    lax.fori_loop(0, MAX_CHUNKS_PER_W, chunk_body, 0)
    plsc.subcore_barrier()

    pltpu.sync_copy(acc.at[pl.ds(s * SLICE, SLICE)], zbuf)
    pltpu.sync_copy(zbuf, out_hbm.at[pl.ds(c * SEG_PAD + s * SLICE, SLICE)])


@functools.partial(
    pl.kernel,
    out_type=jax.ShapeDtypeStruct((NC * SEG_PAD,), jnp.float32),
    mesh=plsc.VectorSubcoreMesh(core_axis_name="c", subcore_axis_name="s",
                                num_cores=NC, num_subcores=NS),
    scratch_types=[
        pltpu.VMEM((CHUNK_ELEMS,), jnp.float32),
        pltpu.VMEM((CHUNK_ELEMS,), jnp.int32),
        pltpu.VMEM_SHARED((SEG_PAD,), jnp.float32),
        pltpu.VMEM((SLICE,), jnp.float32),
    ],
)
def _sc_segment_sum(vals_hbm, idx_hbm, out_hbm, vbuf, ibuf, acc, zbuf):
    _sc_body(vals_hbm, idx_hbm, out_hbm, vbuf, ibuf, acc, zbuf)


def _combine_body(p_ref, o_ref):
    o_ref[...] = p_ref[:SEG_PAD] + p_ref[SEG_PAD:]


def kernel(atom_specific_values, index):
    vals = atom_specific_values
    idx = index.astype(jnp.int32)
    partials = _sc_segment_sum(vals, idx)
    out = pl.pallas_call(
        _combine_body,
        out_shape=jax.ShapeDtypeStruct((SEG_PAD,), jnp.float32),
    )(partials)
    return out[:NSEG]


# speedup vs baseline: 31.8450x; 1.0892x over previous
import jax
import jax.numpy as jnp
from jax import lax
from jax.experimental import pallas as pl
from jax.experimental.pallas import tpu as pltpu, tpu_sc as plsc
import functools

N = 6400000
NSEG = 100000
NC = 2
NS = 16
NW = NC * NS
CHUNK = 16000
NCHUNKS = N // CHUNK
CPW = NCHUNKS // NW + 1
SEG_PAD = 100096
SLICE = SEG_PAD // NS


def _sc_body(vals_hbm, idx_hbm, out_hbm, vbuf0, vbuf1, ibuf0, ibuf1, acc,
             zbuf, vsem0, vsem1, isem0, isem1):
    c = lax.axis_index("c")
    s = lax.axis_index("s")
    wid = s * NC + c
    vbufs = (vbuf0, vbuf1)
    ibufs = (ibuf0, ibuf1)
    vsems = (vsem0, vsem1)
    isems = (isem0, isem1)

    def zfill(i, _):
        zbuf[pl.ds(i * 16, 16)] = jnp.zeros((16,), jnp.float32)
        return 0
    lax.fori_loop(0, SLICE // 16, zfill, 0)
    pltpu.sync_copy(zbuf, acc.at[pl.ds(s * SLICE, SLICE)])

    def issue_load(k):
        b = k % 2
        el0 = jnp.minimum(wid + NW * k, NCHUNKS - 1) * CHUNK
        dv = pltpu.async_copy(vals_hbm.at[pl.ds(el0, CHUNK)],
                              vbufs[b], vsems[b])
        di = pltpu.async_copy(idx_hbm.at[pl.ds(el0, CHUNK)],
                              ibufs[b], isems[b])
        return dv, di

    descs = issue_load(0)
    plsc.subcore_barrier()

    for k in range(CPW):
        b = k % 2
        nxt = issue_load(k + 1) if k + 1 < CPW else None
        dv, di = descs
        dv.wait()
        di.wait()

        @pl.when(wid + NW * k < NCHUNKS)
        def _():
            pltpu.sync_copy(vbufs[b], acc.at[ibufs[b]], add=True)
        descs = nxt
    plsc.subcore_barrier()

    pltpu.sync_copy(acc.at[pl.ds(s * SLICE, SLICE)], zbuf)
    pltpu.sync_copy(zbuf, out_hbm.at[pl.ds(c * SEG_PAD + s * SLICE, SLICE)])


@functools.partial(
    pl.kernel,
    out_type=jax.ShapeDtypeStruct((NC * SEG_PAD,), jnp.float32),
    mesh=plsc.VectorSubcoreMesh(core_axis_name="c", subcore_axis_name="s",
                                num_cores=NC, num_subcores=NS),
    scratch_types=[
        pltpu.VMEM((CHUNK,), jnp.float32),
        pltpu.VMEM((CHUNK,), jnp.float32),
        pltpu.VMEM((CHUNK,), jnp.int32),
        pltpu.VMEM((CHUNK,), jnp.int32),
        pltpu.VMEM_SHARED((SEG_PAD,), jnp.float32),
        pltpu.VMEM((SLICE,), jnp.float32),
        pltpu.SemaphoreType.DMA,
        pltpu.SemaphoreType.DMA,
        pltpu.SemaphoreType.DMA,
        pltpu.SemaphoreType.DMA,
    ],
)
def _sc_segment_sum(vals_hbm, idx_hbm, out_hbm, vbuf0, vbuf1, ibuf0, ibuf1,
                    acc, zbuf, vsem0, vsem1, isem0, isem1):
    _sc_body(vals_hbm, idx_hbm, out_hbm, vbuf0, vbuf1, ibuf0, ibuf1, acc,
             zbuf, vsem0, vsem1, isem0, isem1)


def _combine_body(p_ref, o_ref):
    o_ref[...] = p_ref[:SEG_PAD] + p_ref[SEG_PAD:]


def kernel(atom_specific_values, index):
    vals = atom_specific_values
    idx = index.astype(jnp.int32)
    partials = _sc_segment_sum(vals, idx)
    out = pl.pallas_call(
        _combine_body,
        out_shape=jax.ShapeDtypeStruct((SEG_PAD,), jnp.float32),
    )(partials)
    return out[:NSEG]


# speedup vs baseline: 66.7873x; 2.0973x over previous
import jax
import jax.numpy as jnp
from jax import lax
from jax.experimental import pallas as pl
from jax.experimental.pallas import tpu as pltpu, tpu_sc as plsc
import functools

N = 6400000
NSEG = 100000
NC = 2
NS = 16
NW = NC * NS
APW = N // NW
CHUNK = 2000
CPW = APW // CHUNK
STEPS = CHUNK // 16
LSTRIDE = STEPS
SEG_PAD = 100096
SEG_SP = 102144
SLICE = SEG_SP // NS


def _sc_body(vals_hbm, idx_hbm, out_hbm, vbuf0, vbuf1, ibuf0, ibuf1,
             accl, zstage, acc_sp, vsem0, vsem1, isem0, isem1):
    c = lax.axis_index("c")
    s = lax.axis_index("s")
    wid = s * NC + c
    base_el = wid * APW
    vbufs = (vbuf0, vbuf1)
    ibufs = (ibuf0, ibuf1)
    vsems = (vsem0, vsem1)
    isems = (isem0, isem1)

    def issue_load(k):
        b = k % 2
        el0 = base_el + k * CHUNK
        dv = pltpu.async_copy(vals_hbm.at[pl.ds(el0, CHUNK)],
                              vbufs[b], vsems[b])
        di = pltpu.async_copy(idx_hbm.at[pl.ds(el0, CHUNK)],
                              ibufs[b], isems[b])
        return dv, di

    descs = issue_load(0)

    zvec = jnp.zeros((16,), jnp.float32)

    def zfill_acc(i, _):
        b = i * 128
        for u in range(8):
            accl[pl.ds(b + u * 16, 16)] = zvec
        return 0
    lax.fori_loop(0, SEG_SP // 128, zfill_acc, 0)

    def zfill_st(i, _):
        zstage[pl.ds(i * 16, 16)] = zvec
        return 0
    lax.fori_loop(0, SLICE // 16, zfill_st, 0)
    pltpu.sync_copy(zstage, acc_sp.at[pl.ds(s * SLICE, SLICE)])
    plsc.subcore_barrier()

    loff = lax.iota(jnp.int32, 16) * LSTRIDE

    def do_chunk(vb, ib):
        def step(t, _):
            g = loff + t
            v = plsc.load_gather(vb, [g])
            d = plsc.load_gather(ib, [g])
            plsc.addupdate_scatter(accl, [d], v)
            return 0
        lax.fori_loop(0, STEPS, step, 0)

    dmin = None
    for k in range(CPW):
        b = k % 2
        nxt = issue_load(k + 1) if k + 1 < CPW else None
        dv, di = descs
        dv.wait()
        di.wait()
        if k == 0:
            dmin = ibufs[0][pl.ds(0, 16)][0]
        do_chunk(vbufs[b], ibufs[b])
        descs = nxt

    dmax = ibufs[(CPW - 1) % 2][pl.ds(CHUNK - 16, 16)][15]
    dmin_al = (dmin // 8) * 8
    nf = (dmax - dmin_al) // CHUNK + 1

    def ifill(i, _):
        ibuf0[pl.ds(i * 16, 16)] = lax.iota(jnp.int32, 16) + i * 16
        return 0
    lax.fori_loop(0, STEPS, ifill, 0)

    def flush(f, _):
        fbase = dmin_al + f * CHUNK

        def radd(i, _):
            ibuf1[pl.ds(i * 16, 16)] = ibuf0[pl.ds(i * 16, 16)] + fbase
            return 0
        lax.fori_loop(0, STEPS, radd, 0)
        pltpu.sync_copy(accl.at[pl.ds(fbase, CHUNK)],
                        acc_sp.at[ibuf1], add=True)
        return 0
    lax.fori_loop(0, nf, flush, 0)
    plsc.subcore_barrier()

    pltpu.sync_copy(acc_sp.at[pl.ds(s * SLICE, SLICE)], zstage)
    pltpu.sync_copy(zstage, out_hbm.at[pl.ds(c * SEG_SP + s * SLICE, SLICE)])


@functools.partial(
    pl.kernel,
    out_type=jax.ShapeDtypeStruct((NC * SEG_SP,), jnp.float32),
    mesh=plsc.VectorSubcoreMesh(core_axis_name="c", subcore_axis_name="s",
                                num_cores=NC, num_subcores=NS),
    scratch_types=[
        pltpu.VMEM((CHUNK,), jnp.float32),
        pltpu.VMEM((CHUNK,), jnp.float32),
        pltpu.VMEM((CHUNK,), jnp.int32),
        pltpu.VMEM((CHUNK,), jnp.int32),
        pltpu.VMEM((SEG_SP,), jnp.float32),
        pltpu.VMEM((SLICE,), jnp.float32),
        pltpu.VMEM_SHARED((SEG_SP,), jnp.float32),
        pltpu.SemaphoreType.DMA,
        pltpu.SemaphoreType.DMA,
        pltpu.SemaphoreType.DMA,
        pltpu.SemaphoreType.DMA,
    ],
    compiler_params=pltpu.CompilerParams(needs_layout_passes=False),
)
def _sc_segment_sum(vals_hbm, idx_hbm, out_hbm, vbuf0, vbuf1, ibuf0, ibuf1,
                    accl, zstage, acc_sp, vsem0, vsem1, isem0, isem1):
    _sc_body(vals_hbm, idx_hbm, out_hbm, vbuf0, vbuf1, ibuf0, ibuf1,
             accl, zstage, acc_sp, vsem0, vsem1, isem0, isem1)


def _combine_body(p_ref, o_ref):
    o_ref[...] = (p_ref[pl.ds(0, SEG_PAD)]
                  + p_ref[pl.ds(SEG_SP, SEG_PAD)])


def kernel(atom_specific_values, index):
    vals = atom_specific_values
    idx = index.astype(jnp.int32)
    partials = _sc_segment_sum(vals, idx)
    out = pl.pallas_call(
        _combine_body,
        out_shape=jax.ShapeDtypeStruct((SEG_PAD,), jnp.float32),
    )(partials)
    return out[:NSEG]


# speedup vs baseline: 81.5876x; 1.2216x over previous
import jax
import jax.numpy as jnp
from jax import lax
from jax.experimental import pallas as pl
from jax.experimental.pallas import tpu as pltpu, tpu_sc as plsc
import functools

N = 6400000
NSEG = 100000
NC = 2
NS = 16
NW = NC * NS
APW = N // NW
CHUNK = 2000
CPW = APW // CHUNK
STEPS = CHUNK // 16
LSTRIDE = STEPS
SEG_PAD = 100096
SEG_SP = 102144
SLICE = SEG_SP // NS


def _sc_body(vals_hbm, idx_hbm, out_hbm, vbuf0, vbuf1, ibuf0, ibuf1,
             accl, zstage, acc_sp, vsem0, vsem1, isem0, isem1):
    c = lax.axis_index("c")
    s = lax.axis_index("s")
    wid = s * NC + c
    base_el = wid * APW
    vbufs = (vbuf0, vbuf1)
    ibufs = (ibuf0, ibuf1)
    vsems = (vsem0, vsem1)
    isems = (isem0, isem1)

    def issue_load(k):
        b = k % 2
        el0 = base_el + k * CHUNK
        dv = pltpu.async_copy(vals_hbm.at[pl.ds(el0, CHUNK)],
                              vbufs[b], vsems[b])
        di = pltpu.async_copy(idx_hbm.at[pl.ds(el0, CHUNK)],
                              ibufs[b], isems[b])
        return dv, di

    descs = issue_load(0)

    zvec = jnp.zeros((16,), jnp.float32)

    def zfill_acc(i, _):
        b = i * 128
        for u in range(8):
            accl[pl.ds(b + u * 16, 16)] = zvec
        return 0
    lax.fori_loop(0, SEG_SP // 128, zfill_acc, 0)

    def zfill_st(i, _):
        zstage[pl.ds(i * 16, 16)] = zvec
        return 0
    lax.fori_loop(0, SLICE // 16, zfill_st, 0)
    pltpu.sync_copy(zstage, acc_sp.at[pl.ds(s * SLICE, SLICE)])
    plsc.subcore_barrier()

    loff = lax.iota(jnp.int32, 16) * LSTRIDE

    UNROLL = 5

    def do_chunk(vb, ib):
        def step(tt, _):
            t0 = tt * UNROLL
            gs = [loff + (t0 + u) for u in range(UNROLL)]
            vs = [plsc.load_gather(vb, [g]) for g in gs]
            ds = [plsc.load_gather(ib, [g]) for g in gs]
            for u in range(UNROLL):
                plsc.addupdate_scatter(accl, [ds[u]], vs[u])
            return 0
        lax.fori_loop(0, STEPS // UNROLL, step, 0)

    dmin = None
    for k in range(CPW):
        b = k % 2
        nxt = issue_load(k + 1) if k + 1 < CPW else None
        dv, di = descs
        dv.wait()
        di.wait()
        if k == 0:
            dmin = ibufs[0][pl.ds(0, 16)][0]
        do_chunk(vbufs[b], ibufs[b])
        descs = nxt

    dmax = ibufs[(CPW - 1) % 2][pl.ds(CHUNK - 16, 16)][15]
    dmin_al = (dmin // 8) * 8
    nf = (dmax - dmin_al) // CHUNK + 1

    def ifill(i, _):
        ibuf0[pl.ds(i * 16, 16)] = lax.iota(jnp.int32, 16) + i * 16
        return 0
    lax.fori_loop(0, STEPS, ifill, 0)

    def flush(f, _):
        fbase = dmin_al + f * CHUNK

        def radd(i, _):
            ibuf1[pl.ds(i * 16, 16)] = ibuf0[pl.ds(i * 16, 16)] + fbase
            return 0
        lax.fori_loop(0, STEPS, radd, 0)
        pltpu.sync_copy(accl.at[pl.ds(fbase, CHUNK)],
                        acc_sp.at[ibuf1], add=True)
        return 0
    lax.fori_loop(0, nf, flush, 0)
    plsc.subcore_barrier()

    pltpu.sync_copy(acc_sp.at[pl.ds(s * SLICE, SLICE)], zstage)
    pltpu.sync_copy(zstage, out_hbm.at[pl.ds(c * SEG_SP + s * SLICE, SLICE)])


@functools.partial(
    pl.kernel,
    out_type=jax.ShapeDtypeStruct((NC * SEG_SP,), jnp.float32),
    mesh=plsc.VectorSubcoreMesh(core_axis_name="c", subcore_axis_name="s",
                                num_cores=NC, num_subcores=NS),
    scratch_types=[
        pltpu.VMEM((CHUNK,), jnp.float32),
        pltpu.VMEM((CHUNK,), jnp.float32),
        pltpu.VMEM((CHUNK,), jnp.int32),
        pltpu.VMEM((CHUNK,), jnp.int32),
        pltpu.VMEM((SEG_SP,), jnp.float32),
        pltpu.VMEM((SLICE,), jnp.float32),
        pltpu.VMEM_SHARED((SEG_SP,), jnp.float32),
        pltpu.SemaphoreType.DMA,
        pltpu.SemaphoreType.DMA,
        pltpu.SemaphoreType.DMA,
        pltpu.SemaphoreType.DMA,
    ],
    compiler_params=pltpu.CompilerParams(needs_layout_passes=False),
)
def _sc_segment_sum(vals_hbm, idx_hbm, out_hbm, vbuf0, vbuf1, ibuf0, ibuf1,
                    accl, zstage, acc_sp, vsem0, vsem1, isem0, isem1):
    _sc_body(vals_hbm, idx_hbm, out_hbm, vbuf0, vbuf1, ibuf0, ibuf1,
             accl, zstage, acc_sp, vsem0, vsem1, isem0, isem1)


def _combine_body(p_ref, o_ref):
    o_ref[...] = (p_ref[pl.ds(0, SEG_PAD)]
                  + p_ref[pl.ds(SEG_SP, SEG_PAD)])


def kernel(atom_specific_values, index):
    vals = atom_specific_values
    idx = index.astype(jnp.int32)
    partials = _sc_segment_sum(vals, idx)
    out = pl.pallas_call(
        _combine_body,
        out_shape=jax.ShapeDtypeStruct((SEG_PAD,), jnp.float32),
    )(partials)
    return out[:NSEG]
